# baseline (speedup 1.0000x reference)
import functools

import jax
import jax.numpy as jnp
import numpy as np
from jax import lax
from jax.experimental import pallas as pl
from jax.experimental.pallas import tpu as pltpu
from jax.experimental.pallas import tpu_sc as plsc

N_NODES = 100000
N_EDGES = 3200000

NPAD = 131072
ROWS8 = NPAD // 8
NC, NS = 2, 16
NW = NC * NS
NB_NODES = NPAD // NW
PADROW = NPAD - 1
PADREC = PADROW << 12

EPT = 102400
EPAD = EPT * NW
IDX_ROWS = EPAD // 128
TROWS = EPT // 128
KB = 8
BGROUPS = TROWS // KB

NPB = 3125
_SLOT_OF_NODE = ((np.arange(N_NODES) // NPB) * 4096
                 + np.arange(N_NODES) % NPB).astype(np.int32)
_sl = np.arange(NPAD)
_nl = (_sl >> 12) * NPB + (_sl & 4095)
_NODE_OF_SLOT = np.where((_sl & 4095) < NPB, _nl, N_NODES).astype(np.int32)

CAP = 3968
SEG32 = NW * CAP
LISTS = NW * SEG32 + 256
DUMP = NW * SEG32
PADCHUNK = (LISTS - 128) // 128
TBL = 1040


def _mesh():
    return plsc.VectorSubcoreMesh(core_axis_name="c", subcore_axis_name="s",
                                  num_cores=NC, num_subcores=NS)


def _bkt_body(src_hbm, dst_hbm, lists_hbm, counts_hbm,
              sidx, didx, recbuf, slotbuf, ctr, padrec, sem):
    c = lax.axis_index("c")
    s = lax.axis_index("s")
    p = c * NS + s
    pcap = p * CAP
    ctr[pl.ds(0, 16)] = jnp.zeros((16,), jnp.int32)
    ctr[pl.ds(16, 16)] = jnp.zeros((16,), jnp.int32)
    for j in range(8):
        padrec[pl.ds(j * 16, 16)] = jnp.full((16,), PADREC, jnp.int32)

    @pl.when(jnp.logical_and(c == 0, s == 0))
    def _():
        pltpu.sync_copy(padrec, lists_hbm.at[pl.ds(PADCHUNK * 128, 128)])

    base = p * TROWS

    def body(g, carry):
        row = base + g * KB
        pltpu.sync_copy(src_hbm.at[pl.ds(row, KB)], sidx)
        pltpu.sync_copy(dst_hbm.at[pl.ds(row, KB)], didx)
        for j in range(KB):
            for k in range(8):
                sv = sidx[j, pl.ds(k * 16, 16)]
                dv = didx[j, pl.ds(k * 16, 16)]
                b = jnp.right_shift(dv * 21475, 26)
                dl = dv - b * 3125
                bs = jnp.right_shift(sv * 21475, 26)
                sl = bs * 4096 + (sv - bs * 3125)
                rec = jnp.bitwise_or(lax.shift_left(sl, 12), dl)
                valid = sv < PADROW
                cnt, lastm = plsc.scan_count(b, valid)
                cbase = plsc.load_gather(ctr, [b])
                slot = b * SEG32 + (pcap + (cbase + cnt - 1))
                slot = jnp.where(valid, slot, DUMP)
                recbuf[j, pl.ds(k * 16, 16)] = rec
                slotbuf[j, pl.ds(k * 16, 16)] = slot
                plsc.store_scatter(ctr, [b], cbase + cnt,
                                   mask=jnp.logical_and(lastm, valid))
        for j in range(KB):
            pltpu.sync_copy(recbuf.at[j], lists_hbm.at[slotbuf.at[j]])
        return carry

    lax.fori_loop(0, BGROUPS, body, 0)

    cv0 = ctr[pl.ds(0, 16)]
    cv1 = ctr[pl.ds(16, 16)]
    cr0 = jnp.bitwise_and(cv0 + 127, -128)
    cr1 = jnp.bitwise_and(cv1 + 127, -128)
    for h, (cv, cr) in enumerate(((cv0, cr0), (cv1, cr1))):
        for bi in range(16):
            b = h * 16 + bi
            cb = cv[bi]
            crb = cr[bi]
            segbase = b * SEG32 + pcap
            for i in range(8):
                ii = i * 16 + lax.iota(jnp.int32, 16)
                sl = jnp.where(ii < crb - cb, segbase + cb + ii, DUMP)
                slotbuf[i, pl.ds(0, 16)] = sl
            for i in range(8):
                pltpu.sync_copy(padrec.at[pl.ds(i * 16, 16)],
                                lists_hbm.at[slotbuf.at[i, pl.ds(0, 16)]])
    cidx0 = lax.iota(jnp.int32, 16) * NW + p
    cidx1 = (16 + lax.iota(jnp.int32, 16)) * NW + p
    recbuf[0, pl.ds(0, 16)] = cr0
    recbuf[0, pl.ds(16, 16)] = cr1
    slotbuf[0, pl.ds(0, 16)] = cidx0
    slotbuf[0, pl.ds(16, 16)] = cidx1
    pltpu.sync_copy(recbuf.at[0, pl.ds(0, 16)],
                    counts_hbm.at[slotbuf.at[0, pl.ds(0, 16)]])
    pltpu.sync_copy(recbuf.at[0, pl.ds(16, 16)],
                    counts_hbm.at[slotbuf.at[0, pl.ds(16, 16)]])


@functools.cache
def _bkt_call():
    return pl.kernel(
        _bkt_body,
        out_type=[jax.ShapeDtypeStruct((LISTS,), jnp.int32),
                  jax.ShapeDtypeStruct((NW * NW,), jnp.int32)],
        mesh=_mesh(),
        compiler_params=pltpu.CompilerParams(
            use_tc_tiling_on_sc=False, needs_layout_passes=False),
        scratch_types=[
            pltpu.VMEM((KB, 128), jnp.int32),
            pltpu.VMEM((KB, 128), jnp.int32),
            pltpu.VMEM((KB, 128), jnp.int32),
            pltpu.VMEM((KB, 128), jnp.int32),
            pltpu.VMEM((32,), jnp.int32),
            pltpu.VMEM((128,), jnp.int32),
            pltpu.SemaphoreType.DMA,
        ],
    )


def _build_tbl(counts_hbm, cntv, tbl, w):
    for i in range(TBL // 16):
        tbl[pl.ds(i * 16, 16)] = jnp.full((16,), PADCHUNK, jnp.int32)
    pltpu.sync_copy(counts_hbm.at[pl.ds(w * NW, NW)], cntv)
    cv0 = cntv[pl.ds(0, 16)]
    cv1 = cntv[pl.ds(16, 16)]
    n = jnp.int32(0)
    wbase = w * (SEG32 // 128)
    for h, cv in enumerate((cv0, cv1)):
        for pi in range(16):
            pnum = h * 16 + pi
            nch = jnp.right_shift(cv[pi], 7)
            pbase = wbase + pnum * (CAP // 128)
            for i in range(2):
                ii = i * 16 + lax.iota(jnp.int32, 16)
                plsc.store_compressed(tbl.at[pl.ds(n, 16)], pbase + ii,
                                      mask=ii < nch)
                n = n + jnp.minimum(jnp.maximum(nch - i * 16, 0), 16)
    return n


def _deg_body(lists_hbm, counts_hbm, out_hbm, recs, tbl, cntv, dacc, rsem):
    c = lax.axis_index("c")
    s = lax.axis_index("s")
    w = c * NS + s
    ntot = _build_tbl(counts_hbm, cntv, tbl, w)

    def zbody(g, carry):
        dacc[pl.ds(g * 16, 16)] = jnp.zeros((16,), jnp.float32)
        return carry

    lax.fori_loop(0, NB_NODES // 16, zbody, 0)

    D = 8
    ntot8 = jnp.bitwise_and(ntot + (D - 1), -D)
    tv0 = tbl[pl.ds(0, 16)]
    for k in range(D):
        pltpu.async_copy(lists_hbm.at[pl.ds(tv0[k] * 128, 128)],
                         recs.at[k], rsem.at[k])

    ones = jnp.full((16,), 1.0, jnp.float32)

    def body(v, carry):
        tv = tbl[pl.ds((v + 1) * D, 16)]
        for k in range(D):
            pltpu.make_async_copy(lists_hbm.at[pl.ds(0, 128)], recs.at[k],
                                  rsem.at[k]).wait()
            for j in range(8):
                rv = recs[k, pl.ds(j * 16, 16)]
                dl = jnp.bitwise_and(rv, 4095)
                m = jnp.right_shift(rv, 12) < PADROW
                plsc.addupdate_scatter(dacc, [dl], ones, mask=m)
            pltpu.async_copy(lists_hbm.at[pl.ds(tv[k] * 128, 128)],
                             recs.at[k], rsem.at[k])
        return carry

    nb = jnp.right_shift(ntot8, 3)
    lax.fori_loop(0, nb, body, 0)
    for k in range(D):
        pltpu.make_async_copy(lists_hbm.at[pl.ds(0, 128)], recs.at[k],
                              rsem.at[k]).wait()
    pltpu.sync_copy(dacc, out_hbm.at[pl.ds(w * NB_NODES, NB_NODES)])


@functools.cache
def _deg_call():
    return pl.kernel(
        _deg_body,
        out_type=jax.ShapeDtypeStruct((NPAD,), jnp.float32),
        mesh=_mesh(),
        compiler_params=pltpu.CompilerParams(
            use_tc_tiling_on_sc=False, needs_layout_passes=False),
        scratch_types=[
            pltpu.VMEM((8, 128), jnp.int32),
            pltpu.VMEM((TBL,), jnp.int32),
            pltpu.VMEM((NW,), jnp.int32),
            pltpu.VMEM((NB_NODES,), jnp.float32),
            pltpu.SemaphoreType.DMA((8,)),
        ],
    )


def _agg_body(lists_hbm, counts_hbm, table_hbm, out_hbm,
              recs, dlbuf, sidx, rows, tbl, cntv, acc, rsem, gsem):
    c = lax.axis_index("c")
    s = lax.axis_index("s")
    w = c * NS + s
    ntot = _build_tbl(counts_hbm, cntv, tbl, w)

    def zbody(g, carry):
        acc[g, :] = jnp.zeros((16,), jnp.float32)
        return carry

    lax.fori_loop(0, NB_NODES, zbody, 0)

    D = 8
    ntot8 = jnp.bitwise_and(ntot + (D - 1), -D)

    def unpack_and_gather(k):
        for j in range(8):
            rv = recs[k, pl.ds(j * 16, 16)]
            sidx[k, pl.ds(j * 16, 16)] = jnp.right_shift(rv, 12)
            dlbuf[k, pl.ds(j * 16, 16)] = jnp.bitwise_and(rv, 4095)
        pltpu.async_copy(table_hbm.at[sidx.at[k]], rows.at[k], gsem.at[k])

    tv0 = tbl[pl.ds(0, 16)]
    for k in range(D):
        pltpu.async_copy(lists_hbm.at[pl.ds(tv0[k] * 128, 128)],
                         recs.at[k], rsem.at[k])
    for k in range(D):
        pltpu.make_async_copy(lists_hbm.at[pl.ds(0, 128)], recs.at[k],
                              rsem.at[k]).wait()
        unpack_and_gather(k)
        pltpu.async_copy(lists_hbm.at[pl.ds(tv0[8 + k] * 128, 128)],
                         recs.at[k], rsem.at[k])

    def body(v, carry):
        tv = tbl[pl.ds((v + 2) * D, 16)]
        for k in range(D):
            pltpu.make_async_copy(table_hbm.at[pl.ds(0, 128)], rows.at[k],
                                  gsem.at[k]).wait()
            for j in range(8):
                dv = dlbuf[k, pl.ds(j * 16, 16)]
                for i in range(16):
                    plsc.addupdate(acc.at[dv[i]], rows[k, j * 16 + i, :])
        for k in range(D):
            pltpu.make_async_copy(lists_hbm.at[pl.ds(0, 128)], recs.at[k],
                                  rsem.at[k]).wait()
            unpack_and_gather(k)
            pltpu.async_copy(lists_hbm.at[pl.ds(tv[k] * 128, 128)],
                             recs.at[k], rsem.at[k])
        return carry

    nb = jnp.right_shift(ntot8, 3)
    lax.fori_loop(0, nb, body, 0)
    for k in range(D):
        pltpu.make_async_copy(table_hbm.at[pl.ds(0, 128)], rows.at[k],
                              gsem.at[k]).wait()
        pltpu.make_async_copy(lists_hbm.at[pl.ds(0, 128)], recs.at[k],
                              rsem.at[k]).wait()
    pltpu.sync_copy(acc, out_hbm.at[pl.ds(w * NB_NODES, NB_NODES)])


@functools.cache
def _agg_call():
    return pl.kernel(
        _agg_body,
        out_type=jax.ShapeDtypeStruct((NPAD, 16), jnp.float32),
        mesh=_mesh(),
        compiler_params=pltpu.CompilerParams(
            use_tc_tiling_on_sc=False, needs_layout_passes=False),
        scratch_types=[
            pltpu.VMEM((8, 128), jnp.int32),
            pltpu.VMEM((8, 128), jnp.int32),
            pltpu.VMEM((8, 128), jnp.int32),
            pltpu.VMEM((8, 128, 16), jnp.float32),
            pltpu.VMEM((TBL,), jnp.int32),
            pltpu.VMEM((NW,), jnp.int32),
            pltpu.VMEM((NB_NODES, 16), jnp.float32),
            pltpu.SemaphoreType.DMA((8,)),
            pltpu.SemaphoreType.DMA((8,)),
        ],
    )


def _tc_a(p_ref, x48_ref, w1bd_ref, e8_ref, dinv_ref, hs1_ref):
    deg8 = p_ref[...] + 1.0
    dinv8 = lax.rsqrt(deg8)
    dinv = jnp.dot(dinv8, e8_ref[...], preferred_element_type=jnp.float32)
    dinv_ref[...] = dinv
    xw = jnp.dot(x48_ref[...], w1bd_ref[...], preferred_element_type=jnp.float32)
    hs1_ref[...] = xw * dinv


def _tc_b(q_ref, hs1_ref, dinv_ref, b1t_ref, w2bd_ref, hs2_ref):
    dinv = dinv_ref[...]
    h1 = jnp.maximum(dinv * (q_ref[...] + hs1_ref[...]) + b1t_ref[...], 0.0)
    hs2_ref[...] = jnp.dot(h1, w2bd_ref[...],
                           preferred_element_type=jnp.float32) * dinv


def _tc_c(r_ref, hs2_ref, dinv_ref, b2t_ref, cw1bd_ref, cw2bd_ref,
          cb1t_ref, cb2t_ref, o1_ref, o2_ref):
    dinv = dinv_ref[...]
    h2 = jnp.maximum(dinv * (r_ref[...] + hs2_ref[...]) + b2t_ref[...], 0.0)
    o1_ref[...] = jnp.dot(h2, cw1bd_ref[...],
                          preferred_element_type=jnp.float32) + cb1t_ref[...]
    o2_ref[...] = jnp.dot(h2, cw2bd_ref[...],
                          preferred_element_type=jnp.float32) + cb2t_ref[...]


_tc_a_call = pl.pallas_call(
    _tc_a,
    out_shape=[jax.ShapeDtypeStruct((ROWS8, 128), jnp.float32),
               jax.ShapeDtypeStruct((ROWS8, 128), jnp.float32)],
)

_tc_b_call = pl.pallas_call(
    _tc_b,
    out_shape=jax.ShapeDtypeStruct((ROWS8, 128), jnp.float32),
)

_tc_c_call = pl.pallas_call(
    _tc_c,
    out_shape=[jax.ShapeDtypeStruct((ROWS8, 104), jnp.float32),
               jax.ShapeDtypeStruct((ROWS8, 64), jnp.float32)],
)


def kernel(x, edge_index, W1, b1, W2, b2, CW1, Cb1, CW2, Cb2):
    src = edge_index[0]
    dst = edge_index[1]
    pad_s = jnp.full((EPAD - N_EDGES,), PADROW, jnp.int32)
    pad_d = jnp.zeros((EPAD - N_EDGES,), jnp.int32)
    src_p = jnp.concatenate([src, pad_s]).reshape(IDX_ROWS, 128)
    dst_p = jnp.concatenate([dst, pad_d]).reshape(IDX_ROWS, 128)

    x_slot = jnp.pad(x, ((0, 1), (0, 0)))[jnp.asarray(_NODE_OF_SLOT)]
    x48 = x_slot.reshape(ROWS8, 48)
    eye8 = jnp.eye(8, dtype=jnp.float32)
    w1bd = jnp.kron(eye8, W1)
    w2bd = jnp.kron(eye8, W2)
    cw1bd = jnp.kron(eye8, CW1)
    cw2bd = jnp.kron(eye8, CW2)
    b1t = jnp.tile(b1, 8)[None, :]
    b2t = jnp.tile(b2, 8)[None, :]
    cb1t = jnp.tile(Cb1, 8)[None, :]
    cb2t = jnp.tile(Cb2, 8)[None, :]
    e8 = jnp.repeat(eye8, 16, axis=1)

    lists, counts = _bkt_call()(src_p, dst_p)
    deg = _deg_call()(lists, counts)
    dinv, hs1 = _tc_a_call(deg.reshape(ROWS8, 8), x48, w1bd, e8)
    q = _agg_call()(lists, counts, hs1.reshape(NPAD, 16))
    hs2 = _tc_b_call(q.reshape(ROWS8, 128), hs1, dinv, b1t, w2bd)
    r = _agg_call()(lists, counts, hs2.reshape(NPAD, 16))
    o1p, o2p = _tc_c_call(r.reshape(ROWS8, 128), hs2, dinv, b2t,
                          cw1bd, cw2bd, cb1t, cb2t)
    perm = jnp.asarray(_SLOT_OF_NODE)
    out_1 = o1p.reshape(NPAD, 13)[perm]
    out_2 = o2p.reshape(NPAD, 8)[perm]
    return (out_1, out_2)


# speedup vs baseline: 1.0120x; 1.0120x over previous
import functools

import jax
import jax.numpy as jnp
import numpy as np
from jax import lax
from jax.experimental import pallas as pl
from jax.experimental.pallas import tpu as pltpu
from jax.experimental.pallas import tpu_sc as plsc

N_NODES = 100000
N_EDGES = 3200000

NPAD = 131072
ROWS8 = NPAD // 8
NC, NS = 2, 16
NW = NC * NS
NB_NODES = NPAD // NW
PADROW = NPAD - 1
PADREC = PADROW << 12

EPT = 102400
EPAD = EPT * NW
IDX_ROWS = EPAD // 128
TROWS = EPT // 128
KB = 8
BGROUPS = TROWS // KB

NPB = 3125

CAP = 3968
SEG32 = NW * CAP
LISTS = NW * SEG32 + 256
DUMP = NW * SEG32
PADCHUNK = (LISTS - 128) // 128
TBL = 1040


def _mesh():
    return plsc.VectorSubcoreMesh(core_axis_name="c", subcore_axis_name="s",
                                  num_cores=NC, num_subcores=NS)


def _bkt_body(src_hbm, dst_hbm, lists_hbm, counts_hbm,
              sidx, didx, recbuf, slotbuf, ctr, padrec, sem):
    c = lax.axis_index("c")
    s = lax.axis_index("s")
    p = c * NS + s
    pcap = p * CAP
    ctr[pl.ds(0, 16)] = jnp.zeros((16,), jnp.int32)
    ctr[pl.ds(16, 16)] = jnp.zeros((16,), jnp.int32)
    for j in range(8):
        padrec[pl.ds(j * 16, 16)] = jnp.full((16,), PADREC, jnp.int32)

    @pl.when(jnp.logical_and(c == 0, s == 0))
    def _():
        pltpu.sync_copy(padrec, lists_hbm.at[pl.ds(PADCHUNK * 128, 128)])

    base = p * TROWS

    def body(g, carry):
        row = base + g * KB
        pltpu.sync_copy(src_hbm.at[pl.ds(row, KB)], sidx)
        pltpu.sync_copy(dst_hbm.at[pl.ds(row, KB)], didx)
        for j in range(KB):
            for k in range(8):
                sv = sidx[j, pl.ds(k * 16, 16)]
                dv = didx[j, pl.ds(k * 16, 16)]
                b = jnp.right_shift(dv * 21475, 26)
                dl = dv - b * 3125
                rec = jnp.bitwise_or(lax.shift_left(sv, 12), dl)
                valid = sv < PADROW
                cnt, lastm = plsc.scan_count(b, valid)
                cbase = plsc.load_gather(ctr, [b])
                slot = b * SEG32 + (pcap + (cbase + cnt - 1))
                slot = jnp.where(valid, slot, DUMP)
                recbuf[j, pl.ds(k * 16, 16)] = rec
                slotbuf[j, pl.ds(k * 16, 16)] = slot
                plsc.store_scatter(ctr, [b], cbase + cnt,
                                   mask=jnp.logical_and(lastm, valid))
        descs = [pltpu.async_copy(recbuf.at[j], lists_hbm.at[slotbuf.at[j]],
                                  sem) for j in range(KB)]
        for d in descs:
            d.wait()
        return carry

    lax.fori_loop(0, BGROUPS, body, 0)

    cv0 = ctr[pl.ds(0, 16)]
    cv1 = ctr[pl.ds(16, 16)]
    cr0 = jnp.bitwise_and(cv0 + 127, -128)
    cr1 = jnp.bitwise_and(cv1 + 127, -128)
    for h, (cv, cr) in enumerate(((cv0, cr0), (cv1, cr1))):
        for bi in range(16):
            b = h * 16 + bi
            cb = cv[bi]
            crb = cr[bi]
            segbase = b * SEG32 + pcap
            for i in range(8):
                ii = i * 16 + lax.iota(jnp.int32, 16)
                sl = jnp.where(ii < crb - cb, segbase + cb + ii, DUMP)
                slotbuf[i, pl.ds(0, 16)] = sl
            descs = [pltpu.async_copy(padrec.at[pl.ds(i * 16, 16)],
                                      lists_hbm.at[slotbuf.at[i, pl.ds(0, 16)]],
                                      sem) for i in range(8)]
            for d in descs:
                d.wait()
    cidx0 = lax.iota(jnp.int32, 16) * NW + p
    cidx1 = (16 + lax.iota(jnp.int32, 16)) * NW + p
    recbuf[0, pl.ds(0, 16)] = cr0
    recbuf[0, pl.ds(16, 16)] = cr1
    slotbuf[0, pl.ds(0, 16)] = cidx0
    slotbuf[0, pl.ds(16, 16)] = cidx1
    pltpu.sync_copy(recbuf.at[0, pl.ds(0, 16)],
                    counts_hbm.at[slotbuf.at[0, pl.ds(0, 16)]])
    pltpu.sync_copy(recbuf.at[0, pl.ds(16, 16)],
                    counts_hbm.at[slotbuf.at[0, pl.ds(16, 16)]])


@functools.cache
def _bkt_call():
    return pl.kernel(
        _bkt_body,
        out_type=[jax.ShapeDtypeStruct((LISTS,), jnp.int32),
                  jax.ShapeDtypeStruct((NW * NW,), jnp.int32)],
        mesh=_mesh(),
        compiler_params=pltpu.CompilerParams(
            use_tc_tiling_on_sc=False, needs_layout_passes=False),
        scratch_types=[
            pltpu.VMEM((KB, 128), jnp.int32),
            pltpu.VMEM((KB, 128), jnp.int32),
            pltpu.VMEM((KB, 128), jnp.int32),
            pltpu.VMEM((KB, 128), jnp.int32),
            pltpu.VMEM((32,), jnp.int32),
            pltpu.VMEM((128,), jnp.int32),
            pltpu.SemaphoreType.DMA,
        ],
    )


def _build_tbl(counts_hbm, cntv, tbl, w):
    for i in range(TBL // 16):
        tbl[pl.ds(i * 16, 16)] = jnp.full((16,), PADCHUNK, jnp.int32)
    pltpu.sync_copy(counts_hbm.at[pl.ds(w * NW, NW)], cntv)
    cv0 = cntv[pl.ds(0, 16)]
    cv1 = cntv[pl.ds(16, 16)]
    n = jnp.int32(0)
    wbase = w * (SEG32 // 128)
    for h, cv in enumerate((cv0, cv1)):
        for pi in range(16):
            pnum = h * 16 + pi
            nch = jnp.right_shift(cv[pi], 7)
            pbase = wbase + pnum * (CAP // 128)
            for i in range(2):
                ii = i * 16 + lax.iota(jnp.int32, 16)
                plsc.store_compressed(tbl.at[pl.ds(n, 16)], pbase + ii,
                                      mask=ii < nch)
                n = n + jnp.minimum(jnp.maximum(nch - i * 16, 0), 16)
    return n


def _deg_body(lists_hbm, counts_hbm, out_hbm, recs, tbl, cntv, dacc, rsem):
    c = lax.axis_index("c")
    s = lax.axis_index("s")
    w = c * NS + s
    ntot = _build_tbl(counts_hbm, cntv, tbl, w)

    def zbody(g, carry):
        dacc[pl.ds(g * 16, 16)] = jnp.zeros((16,), jnp.float32)
        return carry

    lax.fori_loop(0, NB_NODES // 16, zbody, 0)

    D = 8
    ntot8 = jnp.bitwise_and(ntot + (D - 1), -D)
    tv0 = tbl[pl.ds(0, 16)]
    for k in range(D):
        pltpu.async_copy(lists_hbm.at[pl.ds(tv0[k] * 128, 128)],
                         recs.at[k], rsem.at[k])

    ones = jnp.full((16,), 1.0, jnp.float32)

    def body(v, carry):
        tv = tbl[pl.ds((v + 1) * D, 16)]
        for k in range(D):
            pltpu.make_async_copy(lists_hbm.at[pl.ds(0, 128)], recs.at[k],
                                  rsem.at[k]).wait()
            for j in range(8):
                rv = recs[k, pl.ds(j * 16, 16)]
                dl = jnp.bitwise_and(rv, 4095)
                m = jnp.right_shift(rv, 12) < PADROW
                plsc.addupdate_scatter(dacc, [dl], ones, mask=m)
            pltpu.async_copy(lists_hbm.at[pl.ds(tv[k] * 128, 128)],
                             recs.at[k], rsem.at[k])
        return carry

    nb = jnp.right_shift(ntot8, 3)
    lax.fori_loop(0, nb, body, 0)
    for k in range(D):
        pltpu.make_async_copy(lists_hbm.at[pl.ds(0, 128)], recs.at[k],
                              rsem.at[k]).wait()
    pltpu.sync_copy(dacc, out_hbm.at[pl.ds(w * NB_NODES, NB_NODES)])


@functools.cache
def _deg_call():
    return pl.kernel(
        _deg_body,
        out_type=jax.ShapeDtypeStruct((NPAD,), jnp.float32),
        mesh=_mesh(),
        compiler_params=pltpu.CompilerParams(
            use_tc_tiling_on_sc=False, needs_layout_passes=False),
        scratch_types=[
            pltpu.VMEM((8, 128), jnp.int32),
            pltpu.VMEM((TBL,), jnp.int32),
            pltpu.VMEM((NW,), jnp.int32),
            pltpu.VMEM((NB_NODES,), jnp.float32),
            pltpu.SemaphoreType.DMA((8,)),
        ],
    )


def _agg_body(lists_hbm, counts_hbm, table_hbm, out_hbm,
              recs, dlbuf, sidx, rows, tbl, cntv, acc, rsem, gsem):
    c = lax.axis_index("c")
    s = lax.axis_index("s")
    w = c * NS + s
    ntot = _build_tbl(counts_hbm, cntv, tbl, w)

    def zbody(g, carry):
        acc[g, :] = jnp.zeros((16,), jnp.float32)
        return carry

    lax.fori_loop(0, NB_NODES, zbody, 0)

    D = 8
    ntot8 = jnp.bitwise_and(ntot + (D - 1), -D)

    def unpack_and_gather(k):
        for j in range(8):
            rv = recs[k, pl.ds(j * 16, 16)]
            sidx[k, pl.ds(j * 16, 16)] = jnp.right_shift(rv, 12)
            dlbuf[k, pl.ds(j * 16, 16)] = jnp.bitwise_and(rv, 4095)
        pltpu.async_copy(table_hbm.at[sidx.at[k]], rows.at[k], gsem.at[k])

    tv0 = tbl[pl.ds(0, 16)]
    for k in range(D):
        pltpu.async_copy(lists_hbm.at[pl.ds(tv0[k] * 128, 128)],
                         recs.at[k], rsem.at[k])
    for k in range(D):
        pltpu.make_async_copy(lists_hbm.at[pl.ds(0, 128)], recs.at[k],
                              rsem.at[k]).wait()
        unpack_and_gather(k)
        pltpu.async_copy(lists_hbm.at[pl.ds(tv0[8 + k] * 128, 128)],
                         recs.at[k], rsem.at[k])

    def body(v, carry):
        tv = tbl[pl.ds((v + 2) * D, 16)]
        for k in range(D):
            pltpu.make_async_copy(table_hbm.at[pl.ds(0, 128)], rows.at[k],
                                  gsem.at[k]).wait()
            for j in range(8):
                dv = dlbuf[k, pl.ds(j * 16, 16)]
                for i in range(16):
                    plsc.addupdate(acc.at[dv[i]], rows[k, j * 16 + i, :])
        for k in range(D):
            pltpu.make_async_copy(lists_hbm.at[pl.ds(0, 128)], recs.at[k],
                                  rsem.at[k]).wait()
            unpack_and_gather(k)
            pltpu.async_copy(lists_hbm.at[pl.ds(tv[k] * 128, 128)],
                             recs.at[k], rsem.at[k])
        return carry

    nb = jnp.right_shift(ntot8, 3)
    lax.fori_loop(0, nb, body, 0)
    for k in range(D):
        pltpu.make_async_copy(table_hbm.at[pl.ds(0, 128)], rows.at[k],
                              gsem.at[k]).wait()
        pltpu.make_async_copy(lists_hbm.at[pl.ds(0, 128)], recs.at[k],
                              rsem.at[k]).wait()
    pltpu.sync_copy(acc, out_hbm.at[pl.ds(w * NB_NODES, NB_NODES)])


@functools.cache
def _agg_call():
    return pl.kernel(
        _agg_body,
        out_type=jax.ShapeDtypeStruct((NPAD, 16), jnp.float32),
        mesh=_mesh(),
        compiler_params=pltpu.CompilerParams(
            use_tc_tiling_on_sc=False, needs_layout_passes=False),
        scratch_types=[
            pltpu.VMEM((8, 128), jnp.int32),
            pltpu.VMEM((8, 128), jnp.int32),
            pltpu.VMEM((8, 128), jnp.int32),
            pltpu.VMEM((8, 128, 16), jnp.float32),
            pltpu.VMEM((TBL,), jnp.int32),
            pltpu.VMEM((NW,), jnp.int32),
            pltpu.VMEM((NB_NODES, 16), jnp.float32),
            pltpu.SemaphoreType.DMA((8,)),
            pltpu.SemaphoreType.DMA((8,)),
        ],
    )


def _tc_a(p_ref, x48_ref, w1bd_ref, e8_ref, dinv_ref, hs1_ref):
    deg8 = p_ref[...] + 1.0
    dinv8 = lax.rsqrt(deg8)
    dinv = jnp.dot(dinv8, e8_ref[...], preferred_element_type=jnp.float32)
    dinv_ref[...] = dinv
    xw = jnp.dot(x48_ref[...], w1bd_ref[...], preferred_element_type=jnp.float32)
    hs1_ref[...] = xw * dinv


def _tc_b(q_ref, hs1_ref, dinv_ref, b1t_ref, w2bd_ref, hs2_ref):
    dinv = dinv_ref[...]
    h1 = jnp.maximum(dinv * (q_ref[...] + hs1_ref[...]) + b1t_ref[...], 0.0)
    hs2_ref[...] = jnp.dot(h1, w2bd_ref[...],
                           preferred_element_type=jnp.float32) * dinv


def _tc_c(r_ref, hs2_ref, dinv_ref, b2t_ref, cw1bd_ref, cw2bd_ref,
          cb1t_ref, cb2t_ref, o1_ref, o2_ref):
    dinv = dinv_ref[...]
    h2 = jnp.maximum(dinv * (r_ref[...] + hs2_ref[...]) + b2t_ref[...], 0.0)
    o1_ref[...] = jnp.dot(h2, cw1bd_ref[...],
                          preferred_element_type=jnp.float32) + cb1t_ref[...]
    o2_ref[...] = jnp.dot(h2, cw2bd_ref[...],
                          preferred_element_type=jnp.float32) + cb2t_ref[...]


_tc_a_call = pl.pallas_call(
    _tc_a,
    out_shape=[jax.ShapeDtypeStruct((ROWS8, 128), jnp.float32),
               jax.ShapeDtypeStruct((ROWS8, 128), jnp.float32)],
)

_tc_b_call = pl.pallas_call(
    _tc_b,
    out_shape=jax.ShapeDtypeStruct((ROWS8, 128), jnp.float32),
)

_tc_c_call = pl.pallas_call(
    _tc_c,
    out_shape=[jax.ShapeDtypeStruct((ROWS8, 104), jnp.float32),
               jax.ShapeDtypeStruct((ROWS8, 64), jnp.float32)],
)


def kernel(x, edge_index, W1, b1, W2, b2, CW1, Cb1, CW2, Cb2):
    src = edge_index[0]
    dst = edge_index[1]
    pad_s = jnp.full((EPAD - N_EDGES,), PADROW, jnp.int32)
    pad_d = jnp.zeros((EPAD - N_EDGES,), jnp.int32)
    src_p = jnp.concatenate([src, pad_s]).reshape(IDX_ROWS, 128)
    dst_p = jnp.concatenate([dst, pad_d]).reshape(IDX_ROWS, 128)

    x48 = jnp.pad(x, ((0, NPAD - N_NODES), (0, 0))).reshape(ROWS8, 48)
    eye8 = jnp.eye(8, dtype=jnp.float32)
    w1bd = jnp.kron(eye8, W1)
    w2bd = jnp.kron(eye8, W2)
    cw1bd = jnp.kron(eye8, CW1)
    cw2bd = jnp.kron(eye8, CW2)
    b1t = jnp.tile(b1, 8)[None, :]
    b2t = jnp.tile(b2, 8)[None, :]
    cb1t = jnp.tile(Cb1, 8)[None, :]
    cb2t = jnp.tile(Cb2, 8)[None, :]
    e8 = jnp.repeat(eye8, 16, axis=1)

    def unslot(a):
        node = a.reshape(NW, 4096, -1)[:, :NPB].reshape(N_NODES, -1)
        return jnp.pad(node, ((0, NPAD - N_NODES), (0, 0)))

    lists, counts = _bkt_call()(src_p, dst_p)
    deg = unslot(_deg_call()(lists, counts)[:, None])[:, 0]
    dinv, hs1 = _tc_a_call(deg.reshape(ROWS8, 8), x48, w1bd, e8)
    q = unslot(_agg_call()(lists, counts, hs1.reshape(NPAD, 16)))
    hs2 = _tc_b_call(q.reshape(ROWS8, 128), hs1, dinv, b1t, w2bd)
    r = unslot(_agg_call()(lists, counts, hs2.reshape(NPAD, 16)))
    o1p, o2p = _tc_c_call(r.reshape(ROWS8, 128), hs2, dinv, b2t,
                          cw1bd, cw2bd, cb1t, cb2t)
    out_1 = o1p.reshape(NPAD, 13)[:N_NODES]
    out_2 = o2p.reshape(NPAD, 8)[:N_NODES]
    return (out_1, out_2)


# speedup vs baseline: 8.9292x; 8.8230x over previous
import functools

import jax
import jax.numpy as jnp
import numpy as np
from jax import lax
from jax.experimental import pallas as pl
from jax.experimental.pallas import tpu as pltpu
from jax.experimental.pallas import tpu_sc as plsc

N_NODES = 100000
N_EDGES = 3200000

NPAD = 131072
ROWS8 = NPAD // 8
NC, NS = 2, 16
NW = NC * NS
NB_NODES = NPAD // NW
PADROW = NPAD - 1
PADREC = PADROW << 12

EPT = 102400
EPAD = EPT * NW
IDX_ROWS = EPAD // 128
TROWS = EPT // 128
KB = 8
BGROUPS = TROWS // KB

NPB = 3125

CAP = 4480
SEG32 = NW * CAP
LISTS = NW * SEG32 + 128
PADCHUNK = (LISTS - 128) // 128
TBL = 1168
SCAP = 1120
ROUNDS = (28, 28, 28, 16)


def _mesh():
    return plsc.VectorSubcoreMesh(core_axis_name="c", subcore_axis_name="s",
                                  num_cores=NC, num_subcores=NS)


def _bkt_body(src_hbm, dst_hbm, lists_hbm, counts_hbm,
              sidx, didx, stag, ctr, tot, padrec, sem):
    c = lax.axis_index("c")
    s = lax.axis_index("s")
    p = c * NS + s
    pcap = p * CAP
    iota = lax.iota(jnp.int32, 16)
    tot[pl.ds(0, 16)] = jnp.zeros((16,), jnp.int32)
    tot[pl.ds(16, 16)] = jnp.zeros((16,), jnp.int32)
    for j in range(8):
        padrec[pl.ds(j * 16, 16)] = jnp.full((16,), PADREC, jnp.int32)

    @pl.when(jnp.logical_and(c == 0, s == 0))
    def _():
        pltpu.sync_copy(padrec, lists_hbm.at[pl.ds(PADCHUNK * 128, 128)])

    base = p * TROWS
    padvec = jnp.full((16,), PADREC, jnp.int32)
    goff = 0
    for r, gr in enumerate(ROUNDS):
        ctr[pl.ds(0, 16)] = jnp.zeros((16,), jnp.int32)
        ctr[pl.ds(16, 16)] = jnp.zeros((16,), jnp.int32)

        def body(g, carry, goff=goff):
            row = base + (goff + g) * KB
            pltpu.sync_copy(src_hbm.at[pl.ds(row, KB)], sidx)
            pltpu.sync_copy(dst_hbm.at[pl.ds(row, KB)], didx)
            for j in range(KB):
                for k in range(8):
                    sv = sidx[j, pl.ds(k * 16, 16)]
                    dv = didx[j, pl.ds(k * 16, 16)]
                    b = jnp.right_shift(dv * 21475, 26)
                    dl = dv - b * 3125
                    rec = jnp.bitwise_or(lax.shift_left(sv, 12), dl)
                    valid = sv < PADROW
                    cnt, lastm = plsc.scan_count(b, valid)
                    cbase = plsc.load_gather(ctr, [b])
                    slot = b * SCAP + (cbase + cnt - 1)
                    plsc.store_scatter(stag, [slot], rec, mask=valid)
                    plsc.store_scatter(ctr, [b], cbase + cnt,
                                       mask=jnp.logical_and(lastm, valid))
            return carry

        lax.fori_loop(0, gr, body, 0)
        goff += gr

        cv0 = ctr[pl.ds(0, 16)]
        cv1 = ctr[pl.ds(16, 16)]
        t0 = tot[pl.ds(0, 16)]
        t1 = tot[pl.ds(16, 16)]
        if r < 3:
            c80 = jnp.bitwise_and(cv0 + 7, -8)
            c81 = jnp.bitwise_and(cv1 + 7, -8)
            npadv = 1
        else:
            c80 = jnp.bitwise_and(t0 + cv0 + 127, -128) - t0
            c81 = jnp.bitwise_and(t1 + cv1 + 127, -128) - t1
            npadv = 8
        descs = []
        for h, (cv, c8, t) in enumerate(((cv0, c80, t0), (cv1, c81, t1))):
            for bi in range(16):
                b = h * 16 + bi
                cb = cv[bi]
                c8b = c8[bi]
                for i in range(npadv):
                    ii = i * 16 + iota
                    plsc.store_scatter(stag, [b * SCAP + cb + ii], padvec,
                                       mask=ii < c8b - cb)
                dsto = pl.multiple_of(b * SEG32 + pcap + t[bi], 8)
                descs.append(pltpu.async_copy(
                    stag.at[pl.ds(b * SCAP, SCAP)],
                    lists_hbm.at[pl.ds(dsto, SCAP)],
                    sem))
        for d in descs:
            d.wait()
        tot[pl.ds(0, 16)] = t0 + c80
        tot[pl.ds(16, 16)] = t1 + c81

    sidx[0, pl.ds(0, 16)] = iota * NW + p
    sidx[0, pl.ds(16, 16)] = (16 + iota) * NW + p
    pltpu.sync_copy(tot, counts_hbm.at[sidx.at[0, pl.ds(0, 32)]])


@functools.cache
def _bkt_call():
    return pl.kernel(
        _bkt_body,
        out_type=[jax.ShapeDtypeStruct((LISTS,), jnp.int32),
                  jax.ShapeDtypeStruct((NW * NW,), jnp.int32)],
        mesh=_mesh(),
        compiler_params=pltpu.CompilerParams(
            use_tc_tiling_on_sc=False, needs_layout_passes=False),
        scratch_types=[
            pltpu.VMEM((KB, 128), jnp.int32),
            pltpu.VMEM((KB, 128), jnp.int32),
            pltpu.VMEM((NW * SCAP,), jnp.int32),
            pltpu.VMEM((32,), jnp.int32),
            pltpu.VMEM((32,), jnp.int32),
            pltpu.VMEM((128,), jnp.int32),
            pltpu.SemaphoreType.DMA,
        ],
    )


def _build_tbl(counts_hbm, cntv, tbl, w):
    for i in range(TBL // 16):
        tbl[pl.ds(i * 16, 16)] = jnp.full((16,), PADCHUNK, jnp.int32)
    pltpu.sync_copy(counts_hbm.at[pl.ds(w * NW, NW)], cntv)
    cv0 = cntv[pl.ds(0, 16)]
    cv1 = cntv[pl.ds(16, 16)]
    n = jnp.int32(0)
    wbase = w * (SEG32 // 128)
    for h, cv in enumerate((cv0, cv1)):
        for pi in range(16):
            pnum = h * 16 + pi
            nch = jnp.right_shift(cv[pi], 7)
            pbase = wbase + pnum * (CAP // 128)
            for i in range(3):
                ii = i * 16 + lax.iota(jnp.int32, 16)
                plsc.store_compressed(tbl.at[pl.ds(n, 16)], pbase + ii,
                                      mask=ii < nch)
                n = n + jnp.minimum(jnp.maximum(nch - i * 16, 0), 16)
    return n


def _deg_body(lists_hbm, counts_hbm, out_hbm, recs, tbl, cntv, dacc, rsem):
    c = lax.axis_index("c")
    s = lax.axis_index("s")
    w = c * NS + s
    ntot = _build_tbl(counts_hbm, cntv, tbl, w)

    def zbody(g, carry):
        dacc[pl.ds(g * 16, 16)] = jnp.zeros((16,), jnp.float32)
        return carry

    lax.fori_loop(0, NB_NODES // 16, zbody, 0)

    D = 8
    ntot8 = jnp.bitwise_and(ntot + (D - 1), -D)
    tv0 = tbl[pl.ds(0, 16)]
    for k in range(D):
        pltpu.async_copy(lists_hbm.at[pl.ds(tv0[k] * 128, 128)],
                         recs.at[k], rsem.at[k])

    ones = jnp.full((16,), 1.0, jnp.float32)

    def body(v, carry):
        tv = tbl[pl.ds((v + 1) * D, 16)]
        for k in range(D):
            pltpu.make_async_copy(lists_hbm.at[pl.ds(0, 128)], recs.at[k],
                                  rsem.at[k]).wait()
            for j in range(8):
                rv = recs[k, pl.ds(j * 16, 16)]
                dl = jnp.bitwise_and(rv, 4095)
                m = jnp.right_shift(rv, 12) < PADROW
                plsc.addupdate_scatter(dacc, [dl], ones, mask=m)
            pltpu.async_copy(lists_hbm.at[pl.ds(tv[k] * 128, 128)],
                             recs.at[k], rsem.at[k])
        return carry

    nb = jnp.right_shift(ntot8, 3)
    lax.fori_loop(0, nb, body, 0)
    for k in range(D):
        pltpu.make_async_copy(lists_hbm.at[pl.ds(0, 128)], recs.at[k],
                              rsem.at[k]).wait()
    pltpu.sync_copy(dacc, out_hbm.at[pl.ds(w * NB_NODES, NB_NODES)])


@functools.cache
def _deg_call():
    return pl.kernel(
        _deg_body,
        out_type=jax.ShapeDtypeStruct((NPAD,), jnp.float32),
        mesh=_mesh(),
        compiler_params=pltpu.CompilerParams(
            use_tc_tiling_on_sc=False, needs_layout_passes=False),
        scratch_types=[
            pltpu.VMEM((8, 128), jnp.int32),
            pltpu.VMEM((TBL,), jnp.int32),
            pltpu.VMEM((NW,), jnp.int32),
            pltpu.VMEM((NB_NODES,), jnp.float32),
            pltpu.SemaphoreType.DMA((8,)),
        ],
    )


def _agg_body(lists_hbm, counts_hbm, table_hbm, out_hbm,
              recs, dlbuf, sidx, rows, tbl, cntv, acc, rsem, gsem):
    c = lax.axis_index("c")
    s = lax.axis_index("s")
    w = c * NS + s
    ntot = _build_tbl(counts_hbm, cntv, tbl, w)

    def zbody(g, carry):
        acc[g, :] = jnp.zeros((16,), jnp.float32)
        return carry

    lax.fori_loop(0, NB_NODES, zbody, 0)

    D = 8
    ntot8 = jnp.bitwise_and(ntot + (D - 1), -D)

    def unpack_and_gather(k):
        for j in range(8):
            rv = recs[k, pl.ds(j * 16, 16)]
            sidx[k, pl.ds(j * 16, 16)] = jnp.right_shift(rv, 12)
            dlbuf[k, pl.ds(j * 16, 16)] = jnp.bitwise_and(rv, 4095)
        pltpu.async_copy(table_hbm.at[sidx.at[k]], rows.at[k], gsem.at[k])

    tv0 = tbl[pl.ds(0, 16)]
    for k in range(D):
        pltpu.async_copy(lists_hbm.at[pl.ds(tv0[k] * 128, 128)],
                         recs.at[k], rsem.at[k])
    for k in range(D):
        pltpu.make_async_copy(lists_hbm.at[pl.ds(0, 128)], recs.at[k],
                              rsem.at[k]).wait()
        unpack_and_gather(k)
        pltpu.async_copy(lists_hbm.at[pl.ds(tv0[8 + k] * 128, 128)],
                         recs.at[k], rsem.at[k])

    def body(v, carry):
        tv = tbl[pl.ds((v + 2) * D, 16)]
        for k in range(D):
            pltpu.make_async_copy(table_hbm.at[pl.ds(0, 128)], rows.at[k],
                                  gsem.at[k]).wait()
            for j in range(8):
                dv = dlbuf[k, pl.ds(j * 16, 16)]
                for i in range(16):
                    plsc.addupdate(acc.at[dv[i]], rows[k, j * 16 + i, :])
        for k in range(D):
            pltpu.make_async_copy(lists_hbm.at[pl.ds(0, 128)], recs.at[k],
                                  rsem.at[k]).wait()
            unpack_and_gather(k)
            pltpu.async_copy(lists_hbm.at[pl.ds(tv[k] * 128, 128)],
                             recs.at[k], rsem.at[k])
        return carry

    nb = jnp.right_shift(ntot8, 3)
    lax.fori_loop(0, nb, body, 0)
    for k in range(D):
        pltpu.make_async_copy(table_hbm.at[pl.ds(0, 128)], rows.at[k],
                              gsem.at[k]).wait()
        pltpu.make_async_copy(lists_hbm.at[pl.ds(0, 128)], recs.at[k],
                              rsem.at[k]).wait()
    pltpu.sync_copy(acc, out_hbm.at[pl.ds(w * NB_NODES, NB_NODES)])


@functools.cache
def _agg_call():
    return pl.kernel(
        _agg_body,
        out_type=jax.ShapeDtypeStruct((NPAD, 16), jnp.float32),
        mesh=_mesh(),
        compiler_params=pltpu.CompilerParams(
            use_tc_tiling_on_sc=False, needs_layout_passes=False),
        scratch_types=[
            pltpu.VMEM((8, 128), jnp.int32),
            pltpu.VMEM((8, 128), jnp.int32),
            pltpu.VMEM((8, 128), jnp.int32),
            pltpu.VMEM((8, 128, 16), jnp.float32),
            pltpu.VMEM((TBL,), jnp.int32),
            pltpu.VMEM((NW,), jnp.int32),
            pltpu.VMEM((NB_NODES, 16), jnp.float32),
            pltpu.SemaphoreType.DMA((8,)),
            pltpu.SemaphoreType.DMA((8,)),
        ],
    )


def _tc_a(p_ref, x48_ref, w1bd_ref, e8_ref, dinv_ref, hs1_ref):
    deg8 = p_ref[...] + 1.0
    dinv8 = lax.rsqrt(deg8)
    dinv = jnp.dot(dinv8, e8_ref[...], preferred_element_type=jnp.float32)
    dinv_ref[...] = dinv
    xw = jnp.dot(x48_ref[...], w1bd_ref[...], preferred_element_type=jnp.float32)
    hs1_ref[...] = xw * dinv


def _tc_b(q_ref, hs1_ref, dinv_ref, b1t_ref, w2bd_ref, hs2_ref):
    dinv = dinv_ref[...]
    h1 = jnp.maximum(dinv * (q_ref[...] + hs1_ref[...]) + b1t_ref[...], 0.0)
    hs2_ref[...] = jnp.dot(h1, w2bd_ref[...],
                           preferred_element_type=jnp.float32) * dinv


def _tc_c(r_ref, hs2_ref, dinv_ref, b2t_ref, cw1bd_ref, cw2bd_ref,
          cb1t_ref, cb2t_ref, o1_ref, o2_ref):
    dinv = dinv_ref[...]
    h2 = jnp.maximum(dinv * (r_ref[...] + hs2_ref[...]) + b2t_ref[...], 0.0)
    o1_ref[...] = jnp.dot(h2, cw1bd_ref[...],
                          preferred_element_type=jnp.float32) + cb1t_ref[...]
    o2_ref[...] = jnp.dot(h2, cw2bd_ref[...],
                          preferred_element_type=jnp.float32) + cb2t_ref[...]


_tc_a_call = pl.pallas_call(
    _tc_a,
    out_shape=[jax.ShapeDtypeStruct((ROWS8, 128), jnp.float32),
               jax.ShapeDtypeStruct((ROWS8, 128), jnp.float32)],
)

_tc_b_call = pl.pallas_call(
    _tc_b,
    out_shape=jax.ShapeDtypeStruct((ROWS8, 128), jnp.float32),
)

_tc_c_call = pl.pallas_call(
    _tc_c,
    out_shape=[jax.ShapeDtypeStruct((ROWS8, 104), jnp.float32),
               jax.ShapeDtypeStruct((ROWS8, 64), jnp.float32)],
)


def kernel(x, edge_index, W1, b1, W2, b2, CW1, Cb1, CW2, Cb2):
    src = edge_index[0]
    dst = edge_index[1]
    pad_s = jnp.full((EPAD - N_EDGES,), PADROW, jnp.int32)
    pad_d = jnp.zeros((EPAD - N_EDGES,), jnp.int32)
    src_p = jnp.concatenate([src, pad_s]).reshape(IDX_ROWS, 128)
    dst_p = jnp.concatenate([dst, pad_d]).reshape(IDX_ROWS, 128)

    x48 = jnp.pad(x, ((0, NPAD - N_NODES), (0, 0))).reshape(ROWS8, 48)
    eye8 = jnp.eye(8, dtype=jnp.float32)
    w1bd = jnp.kron(eye8, W1)
    w2bd = jnp.kron(eye8, W2)
    cw1bd = jnp.kron(eye8, CW1)
    cw2bd = jnp.kron(eye8, CW2)
    b1t = jnp.tile(b1, 8)[None, :]
    b2t = jnp.tile(b2, 8)[None, :]
    cb1t = jnp.tile(Cb1, 8)[None, :]
    cb2t = jnp.tile(Cb2, 8)[None, :]
    e8 = jnp.repeat(eye8, 16, axis=1)

    def unslot(a):
        node = a.reshape(NW, 4096, -1)[:, :NPB].reshape(N_NODES, -1)
        return jnp.pad(node, ((0, NPAD - N_NODES), (0, 0)))

    lists, counts = _bkt_call()(src_p, dst_p)
    deg = unslot(_deg_call()(lists, counts)[:, None])[:, 0]
    dinv, hs1 = _tc_a_call(deg.reshape(ROWS8, 8), x48, w1bd, e8)
    q = unslot(_agg_call()(lists, counts, hs1.reshape(NPAD, 16)))
    hs2 = _tc_b_call(q.reshape(ROWS8, 128), hs1, dinv, b1t, w2bd)
    r = unslot(_agg_call()(lists, counts, hs2.reshape(NPAD, 16)))
    o1p, o2p = _tc_c_call(r.reshape(ROWS8, 128), hs2, dinv, b2t,
                          cw1bd, cw2bd, cb1t, cb2t)
    out_1 = o1p.reshape(NPAD, 13)[:N_NODES]
    out_2 = o2p.reshape(NPAD, 8)[:N_NODES]
    return (out_1, out_2)


# speedup vs baseline: 13.8667x; 1.5530x over previous
import functools

import jax
import jax.numpy as jnp
from jax import lax
from jax.experimental import pallas as pl
from jax.experimental.pallas import tpu as pltpu
from jax.experimental.pallas import tpu_sc as plsc

N_NODES = 100000
N_EDGES = 3200000

NPAD = 100352
ROWS8 = NPAD // 8
NC, NS = 2, 16
NW = NC * NS
EPT = 102400
EPAD = EPT * NW
IDX_ROWS = EPAD // 128
TROWS = EPT // 128
KG = 8
GROUPS = TROWS // KG
KD = 16
DGROUPS = TROWS // KD
SLICE = NPAD // NS


def _deg_body(dst_hbm, zeros1_hbm, out_hbm, acc, didx, ones):
    c = lax.axis_index("c")
    s = lax.axis_index("s")
    w = c * NS + s
    pltpu.sync_copy(zeros1_hbm.at[pl.ds(s * SLICE, SLICE)],
                    acc.at[pl.ds(s * SLICE, SLICE)])
    for i in range(8):
        ones[pl.ds(i * 16, 16)] = jnp.full((16,), 1.0, jnp.float32)
    plsc.subcore_barrier()
    base = w * TROWS

    def body(g, carry):
        row = base + g * KD
        pltpu.sync_copy(dst_hbm.at[pl.ds(row, KD)], didx)
        for j in range(KD):
            pltpu.sync_copy(ones, acc.at[didx.at[j]], add=True)
        return carry

    lax.fori_loop(0, DGROUPS, body, 0)
    plsc.subcore_barrier()
    pltpu.sync_copy(acc.at[pl.ds(s * SLICE, SLICE)],
                    out_hbm.at[pl.ds(c * NPAD + s * SLICE, SLICE)])


@functools.cache
def _deg_call():
    mesh = plsc.VectorSubcoreMesh(core_axis_name="c", subcore_axis_name="s",
                                  num_cores=NC, num_subcores=NS)
    return pl.kernel(
        _deg_body,
        out_type=jax.ShapeDtypeStruct((NC * NPAD,), jnp.float32),
        mesh=mesh,
        compiler_params=pltpu.CompilerParams(use_tc_tiling_on_sc=False),
        scratch_types=[
            pltpu.VMEM_SHARED((NPAD,), jnp.float32),
            pltpu.VMEM((KD, 128), jnp.int32),
            pltpu.VMEM((128,), jnp.float32),
        ],
    )


def _agg_body(src_hbm, dst_hbm, table_hbm, zeros2_hbm, out_hbm,
              acc, sidx, didx, rows, sem):
    c = lax.axis_index("c")
    s = lax.axis_index("s")
    w = c * NS + s
    pltpu.sync_copy(zeros2_hbm.at[pl.ds(s * SLICE, SLICE)],
                    acc.at[pl.ds(s * SLICE, SLICE)])
    plsc.subcore_barrier()
    base = w * TROWS

    def body(g, carry):
        row = base + g * KG
        pltpu.sync_copy(src_hbm.at[pl.ds(row, KG)], sidx)
        pltpu.sync_copy(dst_hbm.at[pl.ds(row, KG)], didx)
        descs = [pltpu.async_copy(table_hbm.at[sidx.at[j]], rows.at[j], sem)
                 for j in range(KG)]
        for d in descs:
            d.wait()
        for j in range(KG):
            pltpu.sync_copy(rows.at[j], acc.at[didx.at[j]], add=True)
        return carry

    lax.fori_loop(0, GROUPS, body, 0)
    plsc.subcore_barrier()
    pltpu.sync_copy(acc.at[pl.ds(s * SLICE, SLICE)],
                    out_hbm.at[pl.ds(c * NPAD + s * SLICE, SLICE)])


@functools.cache
def _agg_call():
    mesh = plsc.VectorSubcoreMesh(core_axis_name="c", subcore_axis_name="s",
                                  num_cores=NC, num_subcores=NS)
    return pl.kernel(
        _agg_body,
        out_type=jax.ShapeDtypeStruct((NC * NPAD, 16), jnp.float32),
        mesh=mesh,
        compiler_params=pltpu.CompilerParams(use_tc_tiling_on_sc=False),
        scratch_types=[
            pltpu.VMEM_SHARED((NPAD, 16), jnp.float32),
            pltpu.VMEM((KG, 128), jnp.int32),
            pltpu.VMEM((KG, 128), jnp.int32),
            pltpu.VMEM((KG, 128, 16), jnp.float32),
            pltpu.SemaphoreType.DMA,
        ],
    )


def _tc_a(p_ref, x48_ref, w1bd_ref, e8_ref, dinv_ref, hs1_ref):
    deg8 = p_ref[0] + p_ref[1] + 1.0
    dinv8 = lax.rsqrt(deg8)
    dinv = jnp.dot(dinv8, e8_ref[...], preferred_element_type=jnp.float32)
    dinv_ref[...] = dinv
    xw = jnp.dot(x48_ref[...], w1bd_ref[...], preferred_element_type=jnp.float32)
    hs1_ref[...] = xw * dinv


def _tc_b(q_ref, hs1_ref, dinv_ref, b1t_ref, w2bd_ref, hs2_ref):
    dinv = dinv_ref[...]
    h1 = jnp.maximum(dinv * (q_ref[0] + q_ref[1] + hs1_ref[...]) + b1t_ref[...],
                     0.0)
    hs2_ref[...] = jnp.dot(h1, w2bd_ref[...],
                           preferred_element_type=jnp.float32) * dinv


def _tc_c(r_ref, hs2_ref, dinv_ref, b2t_ref, cw1bd_ref, cw2bd_ref,
          cb1t_ref, cb2t_ref, o1_ref, o2_ref):
    dinv = dinv_ref[...]
    h2 = jnp.maximum(dinv * (r_ref[0] + r_ref[1] + hs2_ref[...]) + b2t_ref[...],
                     0.0)
    o1_ref[...] = jnp.dot(h2, cw1bd_ref[...],
                          preferred_element_type=jnp.float32) + cb1t_ref[...]
    o2_ref[...] = jnp.dot(h2, cw2bd_ref[...],
                          preferred_element_type=jnp.float32) + cb2t_ref[...]


_tc_a_call = pl.pallas_call(
    _tc_a,
    out_shape=[jax.ShapeDtypeStruct((ROWS8, 128), jnp.float32),
               jax.ShapeDtypeStruct((ROWS8, 128), jnp.float32)],
)

_tc_b_call = pl.pallas_call(
    _tc_b,
    out_shape=jax.ShapeDtypeStruct((ROWS8, 128), jnp.float32),
)

_tc_c_call = pl.pallas_call(
    _tc_c,
    out_shape=[jax.ShapeDtypeStruct((ROWS8, 104), jnp.float32),
               jax.ShapeDtypeStruct((ROWS8, 64), jnp.float32)],
)


def kernel(x, edge_index, W1, b1, W2, b2, CW1, Cb1, CW2, Cb2):
    src = edge_index[0]
    dst = edge_index[1]
    padv = jnp.full((EPAD - N_EDGES,), N_NODES, jnp.int32)
    src_p = jnp.concatenate([src, padv]).reshape(IDX_ROWS, 128)
    dst_p = jnp.concatenate([dst, padv]).reshape(IDX_ROWS, 128)
    zeros1 = jnp.zeros((NPAD,), jnp.float32)
    zeros2 = jnp.zeros((NPAD, 16), jnp.float32)

    x48 = jnp.pad(x, ((0, NPAD - N_NODES), (0, 0))).reshape(ROWS8, 48)
    eye8 = jnp.eye(8, dtype=jnp.float32)
    w1bd = jnp.kron(eye8, W1)
    w2bd = jnp.kron(eye8, W2)
    cw1bd = jnp.kron(eye8, CW1)
    cw2bd = jnp.kron(eye8, CW2)
    b1t = jnp.tile(b1, 8)[None, :]
    b2t = jnp.tile(b2, 8)[None, :]
    cb1t = jnp.tile(Cb1, 8)[None, :]
    cb2t = jnp.tile(Cb2, 8)[None, :]
    e8 = jnp.repeat(eye8, 16, axis=1)

    degp = _deg_call()(dst_p, zeros1).reshape(NC, ROWS8, 8)
    dinv, hs1 = _tc_a_call(degp, x48, w1bd, e8)
    q = _agg_call()(src_p, dst_p, hs1.reshape(NPAD, 16), zeros2)
    hs2 = _tc_b_call(q.reshape(NC, ROWS8, 128), hs1, dinv, b1t, w2bd)
    r = _agg_call()(src_p, dst_p, hs2.reshape(NPAD, 16), zeros2)
    o1p, o2p = _tc_c_call(r.reshape(NC, ROWS8, 128), hs2, dinv, b2t,
                          cw1bd, cw2bd, cb1t, cb2t)
    out_1 = o1p.reshape(NPAD, 13)[:N_NODES]
    out_2 = o2p.reshape(NPAD, 8)[:N_NODES]
    return (out_1, out_2)
